# speedup vs baseline: 3.8574x; 3.8574x over previous
import functools

import jax
import jax.numpy as jnp
from jax import lax
from jax.experimental import pallas as pl
from jax.experimental.pallas import tpu as pltpu
from jax.experimental.pallas import tpu_sc as plsc

N = 10000
E = 320000
H = 128
BN_EPS = 1e-5

NC = 2
NS = 16
NW = NC * NS

XPT = 320
XPAD = NW * XPT
XC = 80
XCHUNKS = XPT // XC

EC = 128
EPT = 10112
ECHUNKS = EPT // EC
EPAD = NW * EPT
NSH = 10240
RPT = NSH // NS
DUMMY = N

_MESH = plsc.VectorSubcoreMesh(core_axis_name="c", subcore_axis_name="s")


def _gather_body(x_hbm, emb_hbm, out_hbm, idx_v, rows_v, sem):
    c = lax.axis_index("c")
    s = lax.axis_index("s")
    base = (s * NC + c) * XPT

    def chunk(k, carry):
        off = base + k * XC
        pltpu.sync_copy(x_hbm.at[pl.ds(off, XC)], idx_v)
        pltpu.async_copy(emb_hbm.at[idx_v], rows_v, sem).wait()
        pltpu.sync_copy(rows_v, out_hbm.at[pl.ds(off, XC)])
        return carry

    lax.fori_loop(0, XCHUNKS, chunk, None)


def _gather_rows(x_pad, emb):
    f = pl.kernel(
        _gather_body,
        mesh=_MESH,
        out_type=jax.ShapeDtypeStruct((XPAD, H), jnp.float32),
        scratch_types=[
            pltpu.VMEM((XC,), jnp.int32),
            pltpu.VMEM((XC, H), jnp.float32),
            pltpu.SemaphoreType.DMA,
        ],
    )
    return f(x_pad, emb)


def _agg_body(src_hbm, dst_hbm, h_hbm, out_hbm, src_v, dst_v, rows_v, agg_sh, sem):
    c = lax.axis_index("c")
    s = lax.axis_index("s")

    def zrow(i, carry):
        for j in range(H // 16):
            rows_v[i, pl.ds(j * 16, 16)] = jnp.zeros((16,), jnp.float32)
        return carry

    lax.fori_loop(0, EC, zrow, None)
    for j in range(RPT // EC):
        pltpu.sync_copy(rows_v, agg_sh.at[pl.ds(s * RPT + j * EC, EC)])
    plsc.subcore_barrier()

    base = (s * NC + c) * EPT

    def chunk(k, carry):
        off = base + k * EC
        pltpu.sync_copy(src_hbm.at[pl.ds(off, EC)], src_v)
        pltpu.sync_copy(dst_hbm.at[pl.ds(off, EC)], dst_v)
        pltpu.async_copy(h_hbm.at[src_v], rows_v, sem).wait()
        pltpu.sync_copy(rows_v, agg_sh.at[dst_v], add=True)
        return carry

    lax.fori_loop(0, ECHUNKS, chunk, None)
    plsc.subcore_barrier()

    for j in range(RPT // EC):
        r0 = s * RPT + j * EC
        pltpu.sync_copy(agg_sh.at[pl.ds(r0, EC)],
                        out_hbm.at[pl.ds(c * NSH + r0, EC)])


def _edge_agg(src_pad, dst_pad, h):
    f = pl.kernel(
        _agg_body,
        mesh=_MESH,
        out_type=jax.ShapeDtypeStruct((NC * NSH, H), jnp.float32),
        scratch_types=[
            pltpu.VMEM((EC,), jnp.int32),
            pltpu.VMEM((EC,), jnp.int32),
            pltpu.VMEM((EC, H), jnp.float32),
            pltpu.VMEM_SHARED((NSH, H), jnp.float32),
            pltpu.SemaphoreType.DMA,
        ],
    )
    return f(src_pad, dst_pad, h)


BR = 1000


def _mlp_body(h_ref, p0_ref, p1_ref, w1_ref, b1_ref, w2_ref, b2_ref,
              sc_ref, be_ref, o_ref):
    z = h_ref[...] + p0_ref[...] + p1_ref[...]
    y = jnp.dot(z, w1_ref[...], preferred_element_type=jnp.float32) + b1_ref[...]
    y = jnp.maximum(y, 0.0)
    y = jnp.dot(y, w2_ref[...], preferred_element_type=jnp.float32) + b2_ref[...]
    y = jnp.maximum(y, 0.0)
    o_ref[...] = y * sc_ref[...] + be_ref[...]


def _mlp(h, p0, p1, w1t, b1, w2t, b2, scale, be):
    row = pl.BlockSpec((BR, H), lambda i: (i, 0))
    full = pl.BlockSpec((H, H), lambda i: (0, 0))
    vec = pl.BlockSpec((1, H), lambda i: (0, 0))
    return pl.pallas_call(
        _mlp_body,
        grid=(N // BR,),
        in_specs=[row, row, row, full, vec, full, vec, vec, vec],
        out_specs=row,
        out_shape=jax.ShapeDtypeStruct((N, H), jnp.float32),
    )(h, p0, p1, w1t, b1, w2t, b2, scale, be)


def _final_body(h1_ref, h2_ref, h3_ref, a1_ref, a2_ref, a3_ref, b1_ref,
                w2_ref, b2_ref, o_ref):
    t = (jnp.dot(h1_ref[...], a1_ref[...], preferred_element_type=jnp.float32)
         + jnp.dot(h2_ref[...], a2_ref[...], preferred_element_type=jnp.float32)
         + jnp.dot(h3_ref[...], a3_ref[...], preferred_element_type=jnp.float32)
         + b1_ref[...])
    t = jnp.maximum(t, 0.0)
    o_ref[...] = jnp.dot(t, w2_ref[...], preferred_element_type=jnp.float32) + b2_ref[...]


def _final(h1, h2, h3, a1, a2, a3, b1, w2p, b2p):
    row = pl.BlockSpec((BR, H), lambda i: (i, 0))
    full = pl.BlockSpec((H, H), lambda i: (0, 0))
    vec = pl.BlockSpec((1, H), lambda i: (0, 0))
    return pl.pallas_call(
        _final_body,
        grid=(N // BR,),
        in_specs=[row, row, row, full, full, full, vec,
                  pl.BlockSpec((H, 8), lambda i: (0, 0)),
                  pl.BlockSpec((1, 8), lambda i: (0, 0))],
        out_specs=pl.BlockSpec((BR, 8), lambda i: (i, 0)),
        out_shape=jax.ShapeDtypeStruct((N, 8), jnp.float32),
    )(h1, h2, h3, a1, a2, a3, b1, w2p, b2p)


def kernel(x, edge_index, emb,
           W1_0, b1_0, W2_0, b2_0, g_0, be_0,
           W1_1, b1_1, W2_1, b2_1, g_1, be_1,
           W1_2, b1_2, W2_2, b2_2, g_2, be_2,
           lin1_W, lin1_b, lin2_W, lin2_b):
    x_pad = jnp.concatenate([x, jnp.zeros((XPAD - N,), jnp.int32)])
    src_pad = jnp.concatenate([edge_index[0], jnp.zeros((EPAD - E,), jnp.int32)])
    dst_pad = jnp.concatenate([edge_index[1], jnp.full((EPAD - E,), DUMMY, jnp.int32)])

    h = _gather_rows(x_pad, emb)[:N]

    hs = []
    for (W1, b1, W2, b2, g, be) in (
        (W1_0, b1_0, W2_0, b2_0, g_0, be_0),
        (W1_1, b1_1, W2_1, b2_1, g_1, be_1),
        (W1_2, b1_2, W2_2, b2_2, g_2, be_2),
    ):
        parts = _edge_agg(src_pad, dst_pad, h)
        scale = (g / jnp.sqrt(1.0 + BN_EPS)).reshape(1, H)
        h = _mlp(h, parts[:N], parts[NSH:NSH + N],
                 W1.T, b1.reshape(1, H), W2.T, b2.reshape(1, H),
                 scale, be.reshape(1, H))
        hs.append(h)

    a1 = lin1_W[:, 0:H].T
    a2 = lin1_W[:, H:2 * H].T
    a3 = lin1_W[:, 2 * H:3 * H].T
    w2p = jnp.zeros((H, 8), jnp.float32).at[:, 0:2].set(lin2_W.T)
    b2p = jnp.zeros((8,), jnp.float32).at[0:2].set(lin2_b).reshape(1, 8)

    o = _final(hs[0], hs[1], hs[2], a1, a2, a3, lin1_b.reshape(1, H), w2p, b2p)
    return o[:, 0:2]
